# baseline (speedup 1.0000x reference)
import functools
import jax
import jax.numpy as jnp
from jax import lax
from jax.experimental import pallas as pl
from jax.experimental.pallas import tpu as pltpu, tpu_sc as plsc

_N = 1000
_ROWS = 51200
_NC, _NS = 2, 16
_NW = _NC * _NS
_RPW = _ROWS // _NW
_CR = 32
_NCH = _RPW // _CR
_CE = _CR * _N
_NBUF = 2


def _sc_body(idx_hbm, out_hbm, idx_v, buf0, buf1, sem0, sem1):
    wid = lax.axis_index("s") * _NC + lax.axis_index("c")
    base_row = wid * _RPW
    pltpu.sync_copy(idx_hbm.at[pl.ds(base_row, _RPW)], idx_v)

    zeros16 = jnp.zeros((16,), jnp.float32)
    ones16 = jnp.ones((16,), jnp.float32)
    lane = lax.iota(jnp.int32, 16)
    sems = (sem0, sem1)
    bufs = (buf0, buf1)

    def zbody(i, carry):
        buf0[pl.ds(i * 16, 16)] = zeros16
        buf1[pl.ds(i * 16, 16)] = zeros16
        return carry
    lax.fori_loop(0, _CE // 16, zbody, 0)

    def scatter_chunk(b, c, vals):
        for r in range(_CR // 16):
            iv = idx_v[pl.ds(c * _CR + r * 16, 16)]
            flat = (lane + r * 16) * _N + iv
            plsc.store_scatter(bufs[b], [flat], vals)

    def fire(b, c):
        dst = out_hbm.at[pl.ds((base_row + c * _CR) * _N, _CE)]
        pltpu.async_copy(bufs[b], dst, sems[b])

    def wait(b):
        pltpu.make_async_copy(
            bufs[b], out_hbm.at[pl.ds(base_row * _N, _CE)], sems[b]
        ).wait()

    for b in range(_NBUF):
        scatter_chunk(b, b, ones16)
        fire(b, b)

    def ring_body(c, carry):
        def step(b):
            wait(b)
            scatter_chunk(b, c - _NBUF, zeros16)
            scatter_chunk(b, c, ones16)
            fire(b, c)

        @pl.when(lax.rem(c, 2) == 0)
        def _():
            step(0)

        @pl.when(lax.rem(c, 2) == 1)
        def _():
            step(1)
        return carry
    lax.fori_loop(_NBUF, _NCH, ring_body, 0)

    for b in range(_NBUF):
        wait(b)


def sc_one_hot(flat_idx_i32):
    mesh = plsc.VectorSubcoreMesh(core_axis_name="c", subcore_axis_name="s")
    k = functools.partial(
        pl.kernel, mesh=mesh,
        compiler_params=pltpu.CompilerParams(needs_layout_passes=False),
        out_type=jax.ShapeDtypeStruct((_ROWS * _N,), jnp.float32),
        scratch_types=[
            pltpu.VMEM((_RPW,), jnp.int32),
            pltpu.VMEM((_CE,), jnp.float32),
            pltpu.VMEM((_CE,), jnp.float32),
            pltpu.SemaphoreType.DMA,
            pltpu.SemaphoreType.DMA,
        ],
    )(_sc_body)
    return k(flat_idx_i32)


def kernel(input, eye):
    n = eye.shape[0]
    flat = input.reshape(-1).astype(jnp.int32)
    out = sc_one_hot(flat)
    return out.reshape(*input.shape, n)
